# Initial kernel scaffold; baseline (speedup 1.0000x reference)
import jax
import jax.numpy as jnp
from jax.experimental import pallas as pl


def kernel(seq_length, table):
    raise NotImplementedError("write your pallas kernel here")


# speedup vs baseline: 41.3597x; 41.3597x over previous
import jax
import jax.numpy as jnp
from jax import lax
from jax.experimental import pallas as pl
from jax.experimental.pallas import tpu as pltpu
from jax.experimental.pallas import tpu_sc as plsc

NUM_HEADS = 16
MAX_DIST = 128
S = 2048
NVAR = 16
VLEN = 4096
LANES = 16
CPAD = 304
RAMP0 = 1904
RIGHT0 = 2144


def _rpb_body(cols_hbm, out_hbm, col_v, vref, sem):
    c = lax.axis_index("c")
    s = lax.axis_index("s")
    h = s

    pltpu.sync_copy(cols_hbm.at[h], col_v)

    left = col_v[pl.ds(0, LANES)]
    right = col_v[pl.ds(CPAD - LANES, LANES)]

    for r in range(NVAR):
        base = r * VLEN

        def fill_left(i, carry, base=base):
            vref[pl.ds(base + i * LANES, LANES)] = left
            return carry

        def fill_right(i, carry, base=base):
            vref[pl.ds(base + RIGHT0 + i * LANES, LANES)] = right
            return carry

        def fill_ramp(i, carry, base=base, r=r):
            vals = col_v[pl.ds(LANES - r + i * LANES, LANES)]
            vref[pl.ds(base + RAMP0 + i * LANES, LANES)] = vals
            return carry

        lax.fori_loop(0, RAMP0 // LANES + 1, fill_left, 0)
        lax.fori_loop(0, (VLEN - RIGHT0) // LANES, fill_right, 0)
        lax.fori_loop(0, 18, fill_ramp, 0)

    g_base = c * 64

    def dma_body(j, carry):
        q0 = (g_base + j) * NVAR
        b0 = 2032 - q0
        handles = []
        for r in range(NVAR):
            src_off = pl.multiple_of(r * VLEN + b0, 16)
            dst_off = pl.multiple_of((h * S + q0 + r) * S, S)
            handles.append(pltpu.async_copy(
                vref.at[pl.ds(src_off, S)],
                out_hbm.at[pl.ds(dst_off, S)],
                sem,
            ))
        for hd in handles:
            hd.wait()
        return carry

    lax.fori_loop(0, 64, dma_body, 0)


def kernel(seq_length, table):
    del seq_length
    cols = jnp.pad(table.T, ((0, 0), (LANES, CPAD - LANES - (2 * MAX_DIST + 1))),
                   mode="edge")
    mesh = plsc.VectorSubcoreMesh(core_axis_name="c", subcore_axis_name="s")
    f = pl.kernel(
        _rpb_body,
        mesh=mesh,
        out_type=jax.ShapeDtypeStruct((NUM_HEADS * S * S,), jnp.float32),
        scratch_types=[
            pltpu.VMEM((CPAD,), jnp.float32),
            pltpu.VMEM((NVAR * VLEN,), jnp.float32),
            pltpu.SemaphoreType.DMA,
        ],
    )
    out = f(cols)
    return out.reshape(1, NUM_HEADS, S, S)
